# baseline (speedup 1.0000x reference)
import jax
import jax.numpy as jnp
import numpy as np
from jax.experimental import pallas as pl
from jax.experimental.pallas import tpu as pltpu

E = 16
H = 256
OUT = 32
OBS = 768
CNN_LAT = 512
LAT = 64 + CNN_LAT
BLK = 512

_f32 = jnp.float32
_bf16 = jnp.bfloat16


def _elu(x):
    return jnp.where(x > 0, x, jnp.exp(x) - 1.0)


def _dot(a, b):
    return jax.lax.dot_general(a, b, (((a.ndim - 1,), (0,)), ((), ())),
                               preferred_element_type=_f32)


def _conv_rows(xrows, M, b, last_f32=False):
    z = jnp.zeros_like(xrows[0])
    out = []
    for oy in range(len(xrows) // 2):
        r2 = xrows[2 * oy + 2] if 2 * oy + 2 < len(xrows) else z
        s = jnp.concatenate([xrows[2 * oy], xrows[2 * oy + 1], r2], axis=-1)
        y = _elu(_dot(s, M) + b)
        out.append(y if last_f32 else y.astype(_bf16))
    return out


def _moe_kernel(op_ref, dep_ref, pW0, pb0, pW1, pb1, pg, pbeta,
                M1, b1, M2, b2, M3, b3, cgp, cbp,
                rW0, rb0, rW1, rb1, rW2, rb2,
                eW0, eb0, eW1, eb1, eW2, eb2, eW3, eb3, out_ref):
    h = _elu(_dot(op_ref[...], pW0[...]) + pb0[...])
    h = _dot(h.astype(_bf16), pW1[...]) + pb1[...]
    m = jnp.mean(h, -1, keepdims=True)
    v = jnp.mean((h - m) ** 2, -1, keepdims=True)
    h = (h - m) * jax.lax.rsqrt(v + 1e-5) * pg[...] + pbeta[...]

    rows = [dep_ref[k] for k in range(32)]
    rows = _conv_rows(rows, M1[...], b1[...])
    rows = _conv_rows(rows, M2[...], b2[...])
    rows = _conv_rows(rows, M3[...], b3[...], last_f32=True)
    z = jnp.concatenate(rows, axis=-1)
    m = jnp.mean(z, -1, keepdims=True)
    v = jnp.mean((z - m) ** 2, -1, keepdims=True)
    zc = (z - m) * jax.lax.rsqrt(v + 1e-5) * cgp[...] + cbp[...]

    lat = jnp.concatenate([h, zc], axis=-1).astype(_bf16)

    r = _elu(_dot(lat, rW0[...]) + rb0[...])
    r = _elu(_dot(r.astype(_bf16), rW1[...]) + rb1[...])
    lg = _dot(r.astype(_bf16), rW2[...]) + rb2[...]
    lg = lg - jnp.max(lg, -1, keepdims=True)
    pe = jnp.exp(lg)
    gate = pe / jnp.sum(pe, -1, keepdims=True)

    e0 = _elu(_dot(lat, eW0[...]) + eb0[...]).astype(_bf16)
    acc = jnp.zeros((BLK, OUT), _f32)
    for e in range(E):
        t = e0[:, e * H:(e + 1) * H]
        t = _elu(_dot(t, eW1[e]) + eb1[e])
        t = _elu(_dot(t.astype(_bf16), eW2[e]) + eb2[e])
        t = _dot(t.astype(_bf16), eW3[e]) + eb3[e]
        acc = acc + gate[:, e:e + 1] * t
    out_ref[...] = acc


def _build_M(cw, w_in, c_in, o_w, c_out):
    dy, ox, dx, ci, co = np.meshgrid(
        np.arange(3), np.arange(o_w), np.arange(3),
        np.arange(c_in), np.arange(c_out), indexing='ij')
    w = 2 * ox + dx
    val = (w < w_in)
    rows = (dy * w_in * c_in + w * c_in + ci)[val]
    cols = (ox * c_out + co)[val]
    vals = cw[co[val], ci[val], dy[val], dx[val]]
    M = jnp.zeros((3 * w_in * c_in, o_w * c_out), _f32)
    return M.at[rows, cols].set(vals)


def _full(shape):
    nd = len(shape)
    return pl.BlockSpec(shape, lambda i, _nd=nd: (0,) * _nd)


def kernel(obs_proprio, obs_depth, pW0, pb0, pW1, pb1, pg, pbeta,
           c1, c1b, c2, c2b, c3, c3b, cg, cbeta,
           rW0, rb0, rW1, rb1, rW2, rb2,
           eW0, eb0, eW1, eb1, eW2, eb2, eW3, eb3):
    n = obs_proprio.shape[0]
    dep = obs_depth.reshape(n, 32, 32).transpose(1, 0, 2).astype(_bf16)
    obs = obs_proprio.astype(_bf16)

    M1 = _build_M(c1, 32, 1, 16, 16).astype(_bf16)
    M2 = _build_M(c2, 16, 16, 8, 32).astype(_bf16)
    M3 = _build_M(c3, 8, 32, 4, 32).astype(_bf16)
    b1 = jnp.tile(c1b, 16).reshape(1, 256)
    b2 = jnp.tile(c2b, 8).reshape(1, 256)
    b3 = jnp.tile(c3b, 4).reshape(1, 128)

    hh, ww, cc = np.meshgrid(np.arange(4), np.arange(4), np.arange(32),
                             indexing='ij')
    perm = (cc * 16 + hh * 4 + ww).reshape(-1)
    cgp = cg[perm].reshape(1, CNN_LAT)
    cbp = cbeta[perm].reshape(1, CNN_LAT)
    rW0p = jnp.concatenate([rW0[:64], rW0[64:][perm]], axis=0).astype(_bf16)
    eW0p = jnp.concatenate([eW0[:, :64], eW0[:, 64:][:, perm]], axis=1)
    eW0p = eW0p.transpose(1, 0, 2).reshape(LAT, E * H).astype(_bf16)
    eb0c = eb0.reshape(1, E * H)

    r2 = lambda a: a.reshape(1, -1)

    grid = (n // BLK,)
    out = pl.pallas_call(
        _moe_kernel,
        grid=grid,
        in_specs=[
            pl.BlockSpec((BLK, OBS), lambda i: (i, 0)),
            pl.BlockSpec((32, BLK, 32), lambda i: (0, i, 0)),
            _full((OBS, 128)), _full((1, 128)),
            _full((128, 64)), _full((1, 64)),
            _full((1, 64)), _full((1, 64)),
            _full((96, 256)), _full((1, 256)),
            _full((768, 256)), _full((1, 256)),
            _full((768, 128)), _full((1, 128)),
            _full((1, CNN_LAT)), _full((1, CNN_LAT)),
            _full((LAT, 128)), _full((1, 128)),
            _full((128, 64)), _full((1, 64)),
            _full((64, E)), _full((1, E)),
            _full((LAT, E * H)), _full((1, E * H)),
            _full((E, H, H)), _full((E, H)),
            _full((E, H, H)), _full((E, H)),
            _full((E, H, OUT)), _full((E, OUT)),
        ],
        out_specs=pl.BlockSpec((BLK, OUT), lambda i: (i, 0)),
        out_shape=jax.ShapeDtypeStruct((n, OUT), _f32),
        compiler_params=pltpu.CompilerParams(
            dimension_semantics=("arbitrary",),
        ),
    )(obs, dep, pW0.astype(_bf16), r2(pb0), pW1.astype(_bf16), r2(pb1),
      r2(pg), r2(pbeta),
      M1, b1, M2, b2, M3, b3, cgp, cbp,
      rW0p, r2(rb0), rW1.astype(_bf16), r2(rb1), rW2.astype(_bf16), r2(rb2),
      eW0p, eb0c, eW1.astype(_bf16), eb1, eW2.astype(_bf16), eb2,
      eW3.astype(_bf16), eb3)
    return out


# speedup vs baseline: 6.7463x; 6.7463x over previous
import jax
import jax.numpy as jnp
import numpy as np
from jax.experimental import pallas as pl
from jax.experimental.pallas import tpu as pltpu

E = 16
H = 256
OUT = 32
OBS = 768
CNN_LAT = 512
LAT = 64 + CNN_LAT
BLK = 512

_f32 = jnp.float32
_bf16 = jnp.bfloat16


def _elu(x):
    return jnp.where(x > 0, x, jnp.exp(x) - 1.0)


def _dot(a, b):
    return jax.lax.dot_general(a, b, (((a.ndim - 1,), (0,)), ((), ())),
                               preferred_element_type=_f32)


def _conv_rows(xrows, M, b, last_f32=False):
    z = jnp.zeros_like(xrows[0])
    out = []
    for oy in range(len(xrows) // 2):
        r2 = xrows[2 * oy + 2] if 2 * oy + 2 < len(xrows) else z
        s = jnp.concatenate([xrows[2 * oy], xrows[2 * oy + 1], r2], axis=-1)
        y = _elu(_dot(s, M) + b)
        out.append(y if last_f32 else y.astype(_bf16))
    return out


def _moe_kernel(op_ref, dep_ref, pW0, pb0, pW1, pb1, pg, pbeta,
                M1, b1, M2, b2, M3, b3, cgp, cbp,
                rW0, rb0, rW1, rb1, rW2, rb2,
                eW0, eb0, eW1, eb1, eW2, eb2, eW3, eb3, out_ref):
    h = _elu(_dot(op_ref[...], pW0[...]) + pb0[...])
    h = _dot(h.astype(_bf16), pW1[...]) + pb1[...]
    m = jnp.mean(h, -1, keepdims=True)
    v = jnp.mean((h - m) ** 2, -1, keepdims=True)
    h = (h - m) * jax.lax.rsqrt(v + 1e-5) * pg[...] + pbeta[...]

    rows = [dep_ref[k] for k in range(32)]
    rows = _conv_rows(rows, M1[...], b1[...])
    rows = _conv_rows(rows, M2[...], b2[...])
    rows = _conv_rows(rows, M3[...], b3[...], last_f32=True)
    z = jnp.concatenate(rows, axis=-1)
    m = jnp.mean(z, -1, keepdims=True)
    v = jnp.mean((z - m) ** 2, -1, keepdims=True)
    zc = (z - m) * jax.lax.rsqrt(v + 1e-5) * cgp[...] + cbp[...]

    lat = jnp.concatenate([h, zc], axis=-1).astype(_bf16)

    r = _elu(_dot(lat, rW0[...]) + rb0[...])
    r = _elu(_dot(r.astype(_bf16), rW1[...]) + rb1[...])
    lg = _dot(r.astype(_bf16), rW2[...]) + rb2[...]
    lg = lg - jnp.max(lg, -1, keepdims=True)
    pe = jnp.exp(lg)
    gate = pe / jnp.sum(pe, -1, keepdims=True)

    e0 = _elu(_dot(lat, eW0[...]) + eb0[...]).astype(_bf16)
    acc = jnp.zeros((BLK, OUT), _f32)
    for e in range(E):
        t = e0[:, e * H:(e + 1) * H]
        t = _elu(_dot(t, eW1[e]) + eb1[e])
        t = _elu(_dot(t.astype(_bf16), eW2[e]) + eb2[e])
        t = _dot(t.astype(_bf16), eW3[e]) + eb3[e]
        acc = acc + gate[:, e:e + 1] * t
    out_ref[...] = acc


def _build_M(cw, w_in, c_in, o_w, c_out):
    bands = []
    for dx in range(3):
        b = np.zeros((w_in, o_w), np.float32)
        for ox in range(o_w):
            if 2 * ox + dx < w_in:
                b[2 * ox + dx, ox] = 1.0
        bands.append(b)
    blocks = []
    for dy in range(3):
        m = jnp.zeros((w_in, c_in, o_w, c_out), _f32)
        for dx in range(3):
            a = cw[:, :, dy, dx].T
            m = m + bands[dx][:, None, :, None] * a[None, :, None, :]
        blocks.append(m.reshape(w_in * c_in, o_w * c_out))
    return jnp.concatenate(blocks, axis=0)


def _full(shape):
    nd = len(shape)
    return pl.BlockSpec(shape, lambda i, _nd=nd: (0,) * _nd)


def kernel(obs_proprio, obs_depth, pW0, pb0, pW1, pb1, pg, pbeta,
           c1, c1b, c2, c2b, c3, c3b, cg, cbeta,
           rW0, rb0, rW1, rb1, rW2, rb2,
           eW0, eb0, eW1, eb1, eW2, eb2, eW3, eb3):
    n = obs_proprio.shape[0]
    dep = obs_depth.reshape(n, 32, 32).transpose(1, 0, 2).astype(_bf16)
    obs = obs_proprio.astype(_bf16)

    M1 = _build_M(c1, 32, 1, 16, 16).astype(_bf16)
    M2 = _build_M(c2, 16, 16, 8, 32).astype(_bf16)
    M3 = _build_M(c3, 8, 32, 4, 32).astype(_bf16)
    b1 = jnp.tile(c1b, 16).reshape(1, 256)
    b2 = jnp.tile(c2b, 8).reshape(1, 256)
    b3 = jnp.tile(c3b, 4).reshape(1, 128)

    def cperm(a):
        tail = a.shape[1:]
        return a.reshape(32, 4, 4, *tail).transpose(1, 2, 0, 3) \
                .reshape(CNN_LAT, *tail)
    cgp = cperm(cg.reshape(CNN_LAT, 1)).reshape(1, CNN_LAT)
    cbp = cperm(cbeta.reshape(CNN_LAT, 1)).reshape(1, CNN_LAT)
    rW0p = jnp.concatenate([rW0[:64], cperm(rW0[64:])], axis=0).astype(_bf16)
    eW0t = eW0.transpose(1, 0, 2).reshape(LAT, E * H)
    eW0p = jnp.concatenate([eW0t[:64], cperm(eW0t[64:])],
                           axis=0).astype(_bf16)
    eb0c = eb0.reshape(1, E * H)

    r2 = lambda a: a.reshape(1, -1)

    grid = (n // BLK,)
    out = pl.pallas_call(
        _moe_kernel,
        grid=grid,
        in_specs=[
            pl.BlockSpec((BLK, OBS), lambda i: (i, 0)),
            pl.BlockSpec((32, BLK, 32), lambda i: (0, i, 0)),
            _full((OBS, 128)), _full((1, 128)),
            _full((128, 64)), _full((1, 64)),
            _full((1, 64)), _full((1, 64)),
            _full((96, 256)), _full((1, 256)),
            _full((768, 256)), _full((1, 256)),
            _full((768, 128)), _full((1, 128)),
            _full((1, CNN_LAT)), _full((1, CNN_LAT)),
            _full((LAT, 128)), _full((1, 128)),
            _full((128, 64)), _full((1, 64)),
            _full((64, E)), _full((1, E)),
            _full((LAT, E * H)), _full((1, E * H)),
            _full((E, H, H)), _full((E, H)),
            _full((E, H, H)), _full((E, H)),
            _full((E, H, OUT)), _full((E, OUT)),
        ],
        out_specs=pl.BlockSpec((BLK, OUT), lambda i: (i, 0)),
        out_shape=jax.ShapeDtypeStruct((n, OUT), _f32),
        compiler_params=pltpu.CompilerParams(
            dimension_semantics=("arbitrary",),
        ),
    )(obs, dep, pW0.astype(_bf16), r2(pb0), pW1.astype(_bf16), r2(pb1),
      r2(pg), r2(pbeta),
      M1, b1, M2, b2, M3, b3, cgp, cbp,
      rW0p, r2(rb0), rW1.astype(_bf16), r2(rb1), rW2.astype(_bf16), r2(rb2),
      eW0p, eb0c, eW1.astype(_bf16), eb1, eW2.astype(_bf16), eb2,
      eW3.astype(_bf16), eb3)
    return out


# speedup vs baseline: 7.4378x; 1.1025x over previous
import jax
import jax.numpy as jnp
import numpy as np
from jax.experimental import pallas as pl
from jax.experimental.pallas import tpu as pltpu

E = 16
H = 256
OUT = 32
OBS = 768
CNN_LAT = 512
LAT = 64 + CNN_LAT
BLK = 512

_f32 = jnp.float32
_bf16 = jnp.bfloat16


def _elu(x):
    return jnp.where(x > 0, x, jnp.exp(x) - 1.0)


def _dot(a, b):
    return jax.lax.dot_general(a, b, (((a.ndim - 1,), (0,)), ((), ())),
                               preferred_element_type=_f32)


def _conv_rows(xrows, M, b, last_f32=False):
    z = jnp.zeros_like(xrows[0])
    out = []
    for oy in range(len(xrows) // 2):
        r2 = xrows[2 * oy + 2] if 2 * oy + 2 < len(xrows) else z
        s = jnp.concatenate([xrows[2 * oy], xrows[2 * oy + 1], r2], axis=-1)
        y = _elu(_dot(s, M) + b)
        out.append(y if last_f32 else y.astype(_bf16))
    return out


def _moe_kernel(op_ref, dep_ref, pW0, pb0, pW1, pb1, pg, pbeta,
                M1, b1, M2, b2, M3, b3, cgp, cbp,
                rW0, rb0, rW1, rb1, rW2, rb2,
                eW0, eb0, eW1, eb1, eW2, eb2, eW3, eb3, out_ref):
    h = _elu(_dot(op_ref[...], pW0[...]) + pb0[...])
    h = _dot(h.astype(_bf16), pW1[...]) + pb1[...]
    m = jnp.mean(h, -1, keepdims=True)
    v = jnp.mean((h - m) ** 2, -1, keepdims=True)
    h = (h - m) * jax.lax.rsqrt(v + 1e-5) * pg[...] + pbeta[...]

    x = dep_ref[...]
    m1 = M1[...]
    b1v = b1[...]
    rows = []
    for oy in range(16):
        if oy < 15:
            y = _dot(x[:, 64 * oy:64 * oy + 96], m1)
        else:
            y = _dot(x[:, 960:1024], m1[:64])
        rows.append(_elu(y + b1v).astype(_bf16))
    rows = _conv_rows(rows, M2[...], b2[...])
    rows = _conv_rows(rows, M3[...], b3[...], last_f32=True)
    z = jnp.concatenate(rows, axis=-1)
    m = jnp.mean(z, -1, keepdims=True)
    v = jnp.mean((z - m) ** 2, -1, keepdims=True)
    zc = (z - m) * jax.lax.rsqrt(v + 1e-5) * cgp[...] + cbp[...]

    lat = jnp.concatenate([h, zc], axis=-1).astype(_bf16)

    r = _elu(_dot(lat, rW0[...]) + rb0[...])
    r = _elu(_dot(r.astype(_bf16), rW1[...]) + rb1[...])
    lg = _dot(r.astype(_bf16), rW2[...]) + rb2[...]
    lg = lg - jnp.max(lg, -1, keepdims=True)
    pe = jnp.exp(lg)
    gate = pe / jnp.sum(pe, -1, keepdims=True)

    e0 = _elu(_dot(lat, eW0[...]) + eb0[...]).astype(_bf16)
    acc = jnp.zeros((BLK, OUT), _f32)
    for e in range(E):
        t = e0[:, e * H:(e + 1) * H]
        t = _elu(_dot(t, eW1[e]) + eb1[e])
        t = _elu(_dot(t.astype(_bf16), eW2[e]) + eb2[e])
        t = _dot(t.astype(_bf16), eW3[e]) + eb3[e]
        acc = acc + gate[:, e:e + 1] * t
    out_ref[...] = acc


def _build_M(cw, w_in, c_in, o_w, c_out):
    bands = []
    for dx in range(3):
        b = np.zeros((w_in, o_w), np.float32)
        for ox in range(o_w):
            if 2 * ox + dx < w_in:
                b[2 * ox + dx, ox] = 1.0
        bands.append(b)
    blocks = []
    for dy in range(3):
        m = jnp.zeros((w_in, c_in, o_w, c_out), _f32)
        for dx in range(3):
            a = cw[:, :, dy, dx].T
            m = m + bands[dx][:, None, :, None] * a[None, :, None, :]
        blocks.append(m.reshape(w_in * c_in, o_w * c_out))
    return jnp.concatenate(blocks, axis=0)


def _full(shape):
    nd = len(shape)
    return pl.BlockSpec(shape, lambda i, _nd=nd: (0,) * _nd)


def kernel(obs_proprio, obs_depth, pW0, pb0, pW1, pb1, pg, pbeta,
           c1, c1b, c2, c2b, c3, c3b, cg, cbeta,
           rW0, rb0, rW1, rb1, rW2, rb2,
           eW0, eb0, eW1, eb1, eW2, eb2, eW3, eb3):
    n = obs_proprio.shape[0]
    dep = obs_depth.reshape(n, 1024).astype(_bf16)
    obs = obs_proprio.astype(_bf16)

    M1 = _build_M(c1, 32, 1, 16, 16).astype(_bf16)
    M2 = _build_M(c2, 16, 16, 8, 32).astype(_bf16)
    M3 = _build_M(c3, 8, 32, 4, 32).astype(_bf16)
    b1 = jnp.tile(c1b, 16).reshape(1, 256)
    b2 = jnp.tile(c2b, 8).reshape(1, 256)
    b3 = jnp.tile(c3b, 4).reshape(1, 128)

    def cperm(a):
        tail = a.shape[1:]
        return a.reshape(32, 4, 4, *tail).transpose(1, 2, 0, 3) \
                .reshape(CNN_LAT, *tail)
    cgp = cperm(cg.reshape(CNN_LAT, 1)).reshape(1, CNN_LAT)
    cbp = cperm(cbeta.reshape(CNN_LAT, 1)).reshape(1, CNN_LAT)
    rW0p = jnp.concatenate([rW0[:64], cperm(rW0[64:])], axis=0).astype(_bf16)
    eW0t = eW0.transpose(1, 0, 2).reshape(LAT, E * H)
    eW0p = jnp.concatenate([eW0t[:64], cperm(eW0t[64:])],
                           axis=0).astype(_bf16)
    eb0c = eb0.reshape(1, E * H)

    r2 = lambda a: a.reshape(1, -1)

    grid = (n // BLK,)
    out = pl.pallas_call(
        _moe_kernel,
        grid=grid,
        in_specs=[
            pl.BlockSpec((BLK, OBS), lambda i: (i, 0)),
            pl.BlockSpec((BLK, 1024), lambda i: (i, 0)),
            _full((OBS, 128)), _full((1, 128)),
            _full((128, 64)), _full((1, 64)),
            _full((1, 64)), _full((1, 64)),
            _full((96, 256)), _full((1, 256)),
            _full((768, 256)), _full((1, 256)),
            _full((768, 128)), _full((1, 128)),
            _full((1, CNN_LAT)), _full((1, CNN_LAT)),
            _full((LAT, 128)), _full((1, 128)),
            _full((128, 64)), _full((1, 64)),
            _full((64, E)), _full((1, E)),
            _full((LAT, E * H)), _full((1, E * H)),
            _full((E, H, H)), _full((E, H)),
            _full((E, H, H)), _full((E, H)),
            _full((E, H, OUT)), _full((E, OUT)),
        ],
        out_specs=pl.BlockSpec((BLK, OUT), lambda i: (i, 0)),
        out_shape=jax.ShapeDtypeStruct((n, OUT), _f32),
        compiler_params=pltpu.CompilerParams(
            dimension_semantics=("arbitrary",),
        ),
    )(obs, dep, pW0.astype(_bf16), r2(pb0), pW1.astype(_bf16), r2(pb1),
      r2(pg), r2(pbeta),
      M1, b1, M2, b2, M3, b3, cgp, cbp,
      rW0p, r2(rb0), rW1.astype(_bf16), r2(rb1), rW2.astype(_bf16), r2(rb2),
      eW0p, eb0c, eW1.astype(_bf16), eb1, eW2.astype(_bf16), eb2,
      eW3.astype(_bf16), eb3)
    return out
